# baseline (speedup 1.0000x reference)
import functools

import jax
import jax.numpy as jnp
from jax import lax
from jax.experimental import pallas as pl
from jax.experimental.pallas import tpu as pltpu
from jax.experimental.pallas import tpu_sc as plsc

_NC = 2
_NS = 16
_K = 48
_DD = 16
_PAD = 80
_SINK = 8


def _linear_relu(x, Wt, b2, blk=2000):
    N, D = x.shape

    def body(x_ref, wt_ref, b_ref, o_ref):
        h = jnp.dot(x_ref[...], wt_ref[...], preferred_element_type=jnp.float32)
        o_ref[...] = jnp.maximum(h + b_ref[...], 0.0)

    return pl.pallas_call(
        body,
        grid=(N // blk,),
        in_specs=[
            pl.BlockSpec((blk, D), lambda i: (i, 0)),
            pl.BlockSpec((D, D), lambda i: (0, 0)),
            pl.BlockSpec((1, D), lambda i: (0, 0)),
        ],
        out_specs=pl.BlockSpec((blk, D), lambda i: (i, 0)),
        out_shape=jax.ShapeDtypeStruct((N, D), jnp.float32),
    )(x, Wt, b2)


def _sc_aggregate(xt, e4, zeros, ones, N, D, chunks):
    mesh = plsc.VectorSubcoreMesh(core_axis_name="c", subcore_axis_name="s",
                                  num_cores=_NC)
    rows_per_tile = (N // _NS) // 8 * 8
    tail_base = rows_per_tile * _NS
    tail_rows = N - tail_base

    @functools.partial(
        pl.kernel,
        mesh=mesh,
        compiler_params=pltpu.CompilerParams(use_tc_tiling_on_sc=False),
        out_type=(
            jax.ShapeDtypeStruct((_NC, N, D), jnp.float32),
            jax.ShapeDtypeStruct((_NC, N, _DD), jnp.float32),
        ),
        scratch_types=[
            pltpu.VMEM_SHARED((N + _SINK, D), jnp.float32),
            pltpu.VMEM_SHARED((N + _SINK, _DD), jnp.float32),
            pltpu.VMEM((chunks, _K), jnp.int32),
            pltpu.VMEM((chunks, _K), jnp.int32),
            pltpu.VMEM((_K, _DD), jnp.float32),
            pltpu.VMEM((_K, D), jnp.float32),
            pltpu.VMEM((_K, D), jnp.float32),
            pltpu.VMEM((_K, D), jnp.float32),
            pltpu.SemaphoreType.DMA,
            pltpu.SemaphoreType.DMA,
            pltpu.SemaphoreType.DMA,
            pltpu.SemaphoreType.DMA,
            pltpu.SemaphoreType.DMA,
            pltpu.SemaphoreType.DMA,
            pltpu.SemaphoreType.DMA,
            pltpu.SemaphoreType.DMA,
            pltpu.SemaphoreType.DMA,
        ],
    )
    def body(xt_hbm, e_hbm, z_hbm, ones_hbm, out_hbm, deg_out_hbm,
             acc, deg, src_v, tgt_v, ones_v, r0, r1, r2,
             g0, g1, g2, s0, s1, s2, d0, d1, d2):
        c = lax.axis_index("c")
        s = lax.axis_index("s")
        wid = s * _NC + c
        rows = pl.ds(s * rows_per_tile, rows_per_tile)
        tail = pl.ds(tail_base, tail_rows)
        pltpu.sync_copy(z_hbm.at[rows], acc.at[rows])
        pltpu.sync_copy(z_hbm.at[rows, pl.ds(0, _DD)], deg.at[rows])

        @pl.when(s == _NS - 1)
        def _():
            pltpu.sync_copy(z_hbm.at[tail], acc.at[tail])
            pltpu.sync_copy(z_hbm.at[tail, pl.ds(0, _DD)], deg.at[tail])

        pltpu.sync_copy(e_hbm.at[0, wid], src_v)
        pltpu.sync_copy(e_hbm.at[1, wid], tgt_v)
        pltpu.sync_copy(ones_hbm, ones_v)
        plsc.subcore_barrier()

        buf = (r0, r1, r2)
        gsem = (g0, g1, g2)
        ssem = (s0, s1, s2)
        dsem = (d0, d1, d2)

        def g_start(j, t):
            pltpu.async_copy(xt_hbm.at[tgt_v.at[j]], buf[t], gsem[t])

        def g_wait(j, t):
            pltpu.make_async_copy(
                xt_hbm.at[tgt_v.at[j]], buf[t], gsem[t]).wait()

        def s_start(j, t):
            pltpu.async_copy(buf[t], acc.at[src_v.at[j]], ssem[t], add=True)
            pltpu.async_copy(ones_v, deg.at[src_v.at[j]], dsem[t], add=True)

        def s_wait(j, t):
            pltpu.make_async_copy(
                buf[t], acc.at[src_v.at[j]], ssem[t]).wait()
            pltpu.make_async_copy(
                ones_v, deg.at[src_v.at[j]], dsem[t]).wait()

        def step(j, t, t2):
            g_wait(j, t)
            s_start(j, t)
            s_wait(j - 1, t2)
            g_start(j + 2, t2)

        g_start(0, 0)
        g_start(1, 1)
        g_wait(0, 0)
        s_start(0, 0)
        g_start(2, 2)
        g_wait(1, 1)
        s_start(1, 1)
        s_wait(0, 0)
        g_start(3, 0)

        n_tri = (chunks - 4) // 3
        rem = (chunks - 4) % 3

        def triple(i, carry):
            j0 = 2 + 3 * i
            step(j0, 2, 1)
            step(j0 + 1, 0, 2)
            step(j0 + 2, 1, 0)
            return carry

        lax.fori_loop(0, n_tri, triple, 0)
        for r in range(rem):
            j = 2 + 3 * n_tri + r
            step(j, j % 3, (j + 2) % 3)

        for j in (chunks - 2, chunks - 1):
            g_wait(j, j % 3)
            s_start(j, j % 3)
        for j in (chunks - 3, chunks - 2, chunks - 1):
            s_wait(j, j % 3)

        plsc.subcore_barrier()
        pltpu.sync_copy(acc.at[rows], out_hbm.at[c, rows])
        pltpu.sync_copy(deg.at[rows], deg_out_hbm.at[c, rows])

        @pl.when(s == _NS - 1)
        def _():
            pltpu.sync_copy(acc.at[tail], out_hbm.at[c, tail])
            pltpu.sync_copy(deg.at[tail], deg_out_hbm.at[c, tail])

    return body(xt, e4, zeros, ones)


def _combine(feat, degp, term, N, D, blk=2000):
    def body(p_ref, d_ref, t_ref, o_ref):
        sacc = p_ref[0] + p_ref[1]
        deg = jnp.maximum(d_ref[0][:, :1] + d_ref[1][:, :1], 1.0)
        o_ref[...] = sacc / deg + t_ref[0, 0]

    return pl.pallas_call(
        body,
        grid=(N // blk,),
        in_specs=[
            pl.BlockSpec((2, blk, D), lambda i: (0, i, 0)),
            pl.BlockSpec((2, blk, _DD), lambda i: (0, i, 0)),
            pl.BlockSpec((1, 1), lambda i: (0, 0)),
        ],
        out_specs=pl.BlockSpec((blk, D), lambda i: (i, 0)),
        out_shape=jax.ShapeDtypeStruct((N, D), jnp.float32),
    )(feat, degp, term)


def kernel(x, edge_index, num_node, W, b):
    N, D = x.shape
    E = edge_index.shape[1]
    nw = _NC * _NS
    chunks = (E // nw + _PAD) // _K

    xt = _linear_relu(x, W.T, b.reshape(1, D))
    e2 = edge_index.reshape(2, nw, E // nw)
    sink = N + (jnp.arange(_PAD, dtype=jnp.int32) % _SINK)
    pad = jnp.stack([jnp.broadcast_to(sink, (nw, _PAD)),
                     jnp.zeros((nw, _PAD), jnp.int32)])
    e4 = jnp.concatenate([e2, pad], axis=2).reshape(2, nw, chunks, _K)
    zeros = jnp.zeros((N, D), jnp.float32)
    ones = jnp.ones((_K, _DD), jnp.float32)
    (feat, degp) = _sc_aggregate(xt, e4, zeros, ones, N, D, chunks)
    term = (jnp.asarray(num_node, jnp.float32) - jnp.float32(N)).reshape(1, 1)
    return _combine(feat, degp, term, N, D)


# speedup vs baseline: 1.4394x; 1.4394x over previous
import functools

import jax
import jax.numpy as jnp
from jax import lax
from jax.experimental import pallas as pl
from jax.experimental.pallas import tpu as pltpu
from jax.experimental.pallas import tpu_sc as plsc

_NC = 2
_NS = 16
_K = 40
_DD = 16


def _linear_relu(x, Wt, b2, blk=2000):
    N, D = x.shape

    def body(x_ref, wt_ref, b_ref, o_ref):
        h = jnp.dot(x_ref[...], wt_ref[...], preferred_element_type=jnp.float32)
        o_ref[...] = jnp.maximum(h + b_ref[...], 0.0)

    return pl.pallas_call(
        body,
        grid=(N // blk,),
        in_specs=[
            pl.BlockSpec((blk, D), lambda i: (i, 0)),
            pl.BlockSpec((D, D), lambda i: (0, 0)),
            pl.BlockSpec((1, D), lambda i: (0, 0)),
        ],
        out_specs=pl.BlockSpec((blk, D), lambda i: (i, 0)),
        out_shape=jax.ShapeDtypeStruct((N, D), jnp.float32),
    )(x, Wt, b2)


def _sc_aggregate(xt, e4, zeros, ones, N, D, chunks):
    mesh = plsc.VectorSubcoreMesh(core_axis_name="c", subcore_axis_name="s",
                                  num_cores=_NC)
    rows_per_tile = (N // _NS) // 8 * 8
    tail_base = rows_per_tile * _NS
    tail_rows = N - tail_base

    @functools.partial(
        pl.kernel,
        mesh=mesh,
        compiler_params=pltpu.CompilerParams(use_tc_tiling_on_sc=False),
        out_type=(
            jax.ShapeDtypeStruct((_NC, N, D), jnp.float32),
            jax.ShapeDtypeStruct((_NC, N, _DD), jnp.float32),
        ),
        scratch_types=[
            pltpu.VMEM_SHARED((N, D), jnp.float32),
            pltpu.VMEM_SHARED((N, _DD), jnp.float32),
            pltpu.VMEM((chunks, _K), jnp.int32),
            pltpu.VMEM((chunks, _K), jnp.int32),
            pltpu.VMEM((_K, _DD), jnp.float32),
            pltpu.VMEM((_K, D), jnp.float32),
            pltpu.VMEM((_K, D), jnp.float32),
            pltpu.VMEM((_K, D), jnp.float32),
            pltpu.SemaphoreType.DMA,
            pltpu.SemaphoreType.DMA,
            pltpu.SemaphoreType.DMA,
            pltpu.SemaphoreType.DMA,
            pltpu.SemaphoreType.DMA,
            pltpu.SemaphoreType.DMA,
            pltpu.SemaphoreType.DMA,
            pltpu.SemaphoreType.DMA,
            pltpu.SemaphoreType.DMA,
        ],
    )
    def body(xt_hbm, e_hbm, z_hbm, ones_hbm, out_hbm, deg_out_hbm,
             acc, deg, src_v, tgt_v, ones_v, r0, r1, r2,
             g0, g1, g2, s0, s1, s2, d0, d1, d2):
        c = lax.axis_index("c")
        s = lax.axis_index("s")
        wid = s * _NC + c
        rows = pl.ds(s * rows_per_tile, rows_per_tile)
        tail = pl.ds(tail_base, tail_rows)
        pltpu.sync_copy(z_hbm.at[rows], acc.at[rows])
        pltpu.sync_copy(z_hbm.at[rows, pl.ds(0, _DD)], deg.at[rows])

        @pl.when(s == _NS - 1)
        def _():
            pltpu.sync_copy(z_hbm.at[tail], acc.at[tail])
            pltpu.sync_copy(z_hbm.at[tail, pl.ds(0, _DD)], deg.at[tail])

        pltpu.sync_copy(e_hbm.at[0, wid], src_v)
        pltpu.sync_copy(e_hbm.at[1, wid], tgt_v)
        pltpu.sync_copy(ones_hbm, ones_v)
        plsc.subcore_barrier()

        buf = (r0, r1, r2)
        gsem = (g0, g1, g2)
        ssem = (s0, s1, s2)
        dsem = (d0, d1, d2)

        def g_start(j, t):
            pltpu.async_copy(xt_hbm.at[tgt_v.at[j]], buf[t], gsem[t])

        def g_wait(j, t):
            pltpu.make_async_copy(
                xt_hbm.at[tgt_v.at[j]], buf[t], gsem[t]).wait()

        def s_start(j, t):
            pltpu.async_copy(buf[t], acc.at[src_v.at[j]], ssem[t], add=True)
            pltpu.async_copy(ones_v, deg.at[src_v.at[j]], dsem[t], add=True)

        def s_wait(j, t):
            pltpu.make_async_copy(
                buf[t], acc.at[src_v.at[j]], ssem[t]).wait()
            pltpu.make_async_copy(
                ones_v, deg.at[src_v.at[j]], dsem[t]).wait()

        def step(j, t, t2):
            g_wait(j, t)
            s_start(j, t)
            s_wait(j - 1, t2)
            g_start(j + 2, t2)

        g_start(0, 0)
        g_start(1, 1)
        g_wait(0, 0)
        s_start(0, 0)
        g_start(2, 2)
        g_wait(1, 1)
        s_start(1, 1)
        s_wait(0, 0)
        g_start(3, 0)

        n_tri = (chunks - 4) // 3
        rem = (chunks - 4) % 3

        def triple(i, carry):
            j0 = 2 + 3 * i
            step(j0, 2, 1)
            step(j0 + 1, 0, 2)
            step(j0 + 2, 1, 0)
            return carry

        lax.fori_loop(0, n_tri, triple, 0)
        for r in range(rem):
            j = 2 + 3 * n_tri + r
            step(j, j % 3, (j + 2) % 3)

        for j in (chunks - 2, chunks - 1):
            g_wait(j, j % 3)
            s_start(j, j % 3)
        for j in (chunks - 3, chunks - 2, chunks - 1):
            s_wait(j, j % 3)

        plsc.subcore_barrier()
        pltpu.sync_copy(acc.at[rows], out_hbm.at[c, rows])
        pltpu.sync_copy(deg.at[rows], deg_out_hbm.at[c, rows])

        @pl.when(s == _NS - 1)
        def _():
            pltpu.sync_copy(acc.at[tail], out_hbm.at[c, tail])
            pltpu.sync_copy(deg.at[tail], deg_out_hbm.at[c, tail])

    return body(xt, e4, zeros, ones)


def _combine(feat, degp, term, N, D, blk=2000):
    def body(p_ref, d_ref, t_ref, o_ref):
        sacc = p_ref[0] + p_ref[1]
        deg = jnp.maximum(d_ref[0][:, :1] + d_ref[1][:, :1], 1.0)
        o_ref[...] = sacc / deg + t_ref[0, 0]

    return pl.pallas_call(
        body,
        grid=(N // blk,),
        in_specs=[
            pl.BlockSpec((2, blk, D), lambda i: (0, i, 0)),
            pl.BlockSpec((2, blk, _DD), lambda i: (0, i, 0)),
            pl.BlockSpec((1, 1), lambda i: (0, 0)),
        ],
        out_specs=pl.BlockSpec((blk, D), lambda i: (i, 0)),
        out_shape=jax.ShapeDtypeStruct((N, D), jnp.float32),
    )(feat, degp, term)


def kernel(x, edge_index, num_node, W, b):
    N, D = x.shape
    E = edge_index.shape[1]
    nw = _NC * _NS
    chunks = E // (nw * _K)

    xt = _linear_relu(x, W.T, b.reshape(1, D))
    e4 = edge_index.reshape(2, nw, chunks, _K)
    zeros = jnp.zeros((N, D), jnp.float32)
    ones = jnp.ones((_K, _DD), jnp.float32)
    (feat, degp) = _sc_aggregate(xt, e4, zeros, ones, N, D, chunks)
    term = (jnp.asarray(num_node, jnp.float32) - jnp.float32(N)).reshape(1, 1)
    return _combine(feat, degp, term, N, D)
